# Initial kernel scaffold; baseline (speedup 1.0000x reference)
import jax
import jax.numpy as jnp
from jax.experimental import pallas as pl


def kernel(params, x_Path, x_DNS_Host, x_Package_Name, x_IP, x_Hostnames, x_Command, x_Port, edge_index_Action_Path, edge_index_DNS_DNSHost, edge_index_CMD_Command, edge_index_Socket_IP, edge_index_Socket_Port, edge_index_Socket_Hostnames):
    raise NotImplementedError("write your pallas kernel here")


# speedup vs baseline: 16341.2100x; 16341.2100x over previous
import jax
import jax.numpy as jnp
from jax.experimental import pallas as pl
from jax.experimental.pallas import tpu as pltpu

_NODE_TYPES = ['Path', 'DNS Host', 'Package_Name', 'IP', 'Hostnames', 'Command', 'Port']
_EDGE_KEYS = ['Action_Path', 'DNS_DNSHost', 'CMD_Command', 'Socket_IP', 'Socket_Port', 'Socket_Hostnames']
_TGT_OF_REL = ['Path', 'DNS Host', 'Command', 'IP', 'Port', 'Hostnames']
_PN_IDX = _NODE_TYPES.index('Package_Name')
_NUM_NT = len(_NODE_TYPES)
_NUM_REL = len(_EDGE_KEYS)
_HC = 256
_K = 16


def _loss_kernel(b_ref, s_ref, out_ref):
    rows = []
    for n in range(_NUM_NT):
        z = jnp.sum(b_ref[n, :][:, None] * s_ref[n], axis=0)
        rows.append(z[None, :])
    z_all = jnp.concatenate(rows, axis=0)
    z_max = jnp.max(z_all, axis=-1, keepdims=True)
    ez = jnp.exp(z_all - z_max)
    p = ez / jnp.sum(ez, axis=-1, keepdims=True)

    p_src = p[_PN_IDX]
    link = 0.0
    for r in range(_NUM_REL):
        t = _NODE_TYPES.index(_TGT_OF_REL[r])
        pred = jnp.sum(p_src * p[t])
        link = link + (1.0 - pred) ** 2

    ent_rows = -jnp.sum(p * jnp.log(p + 1e-15), axis=-1)
    ent = jnp.sum(ent_rows) / _NUM_NT

    out_ref[:, :] = jnp.reshape(link + ent, (1, 1))


def kernel(params, x_Path, x_DNS_Host, x_Package_Name, x_IP, x_Hostnames,
           x_Command, x_Port, edge_index_Action_Path, edge_index_DNS_DNSHost,
           edge_index_CMD_Command, edge_index_Socket_IP,
           edge_index_Socket_Port, edge_index_Socket_Hostnames):
    bias_of_nt = {tt: params['conv2_' + ek]['bias']
                  for ek, tt in zip(_EDGE_KEYS, _TGT_OF_REL)}
    b7 = jnp.stack([bias_of_nt.get(nt, jnp.zeros((_HC,), jnp.float32))
                    for nt in _NODE_TYPES])
    s7 = jnp.stack([params['s_' + nt.replace(' ', '_')]
                    for nt in _NODE_TYPES])

    out = pl.pallas_call(
        _loss_kernel,
        out_shape=jax.ShapeDtypeStruct((1, 1), jnp.float32),
    )(b7, s7)
    return out[0, 0]


# speedup vs baseline: 18280.8425x; 1.1187x over previous
import jax
import jax.numpy as jnp
from jax.experimental import pallas as pl

_NODE_TYPES = ['Path', 'DNS Host', 'Package_Name', 'IP', 'Hostnames', 'Command', 'Port']
_EDGE_KEYS = ['Action_Path', 'DNS_DNSHost', 'CMD_Command', 'Socket_IP', 'Socket_Port', 'Socket_Hostnames']
_TGT_OF_REL = ['Path', 'DNS Host', 'Command', 'IP', 'Port', 'Hostnames']
_REL_OF_NT = {tt: r for r, tt in enumerate(_TGT_OF_REL)}
_PN_IDX = _NODE_TYPES.index('Package_Name')
_NUM_NT = len(_NODE_TYPES)
_NUM_REL = len(_EDGE_KEYS)
_HC = 256
_K = 16


def _loss_kernel(*refs):
    bias_refs = refs[:_NUM_REL]
    s_refs = refs[_NUM_REL:_NUM_REL + _NUM_NT]
    out_ref = refs[-1]

    rows = []
    for n, nt in enumerate(_NODE_TYPES):
        if nt in _REL_OF_NT:
            b = bias_refs[_REL_OF_NT[nt]][0, :]
            z = jnp.sum(b[:, None] * s_refs[n][...], axis=0)[None, :]
        else:
            z = jnp.zeros((1, _K), jnp.float32)
        rows.append(z)
    z_all = jnp.concatenate(rows, axis=0)

    z_max = jnp.max(z_all, axis=-1, keepdims=True)
    ez = jnp.exp(z_all - z_max)
    p = ez / jnp.sum(ez, axis=-1, keepdims=True)

    p_src = p[_PN_IDX]
    link = 0.0
    for r in range(_NUM_REL):
        t = _NODE_TYPES.index(_TGT_OF_REL[r])
        pred = jnp.sum(p_src * p[t])
        link = link + (1.0 - pred) ** 2

    ent_rows = -jnp.sum(p * jnp.log(p + 1e-15), axis=-1)
    ent = jnp.sum(ent_rows) / _NUM_NT

    out_ref[:, :] = jnp.reshape(link + ent, (1, 1))


def kernel(params, x_Path, x_DNS_Host, x_Package_Name, x_IP, x_Hostnames,
           x_Command, x_Port, edge_index_Action_Path, edge_index_DNS_DNSHost,
           edge_index_CMD_Command, edge_index_Socket_IP,
           edge_index_Socket_Port, edge_index_Socket_Hostnames):
    biases = [params['conv2_' + ek]['bias'].reshape(1, _HC) for ek in _EDGE_KEYS]
    s_mats = [params['s_' + nt.replace(' ', '_')] for nt in _NODE_TYPES]

    out = pl.pallas_call(
        _loss_kernel,
        out_shape=jax.ShapeDtypeStruct((1, 1), jnp.float32),
    )(*biases, *s_mats)
    return out[0, 0]
